# baseline (speedup 1.0000x reference)
import functools

import jax
import jax.numpy as jnp
from jax.experimental import pallas as pl
from jax.experimental.pallas import tpu as pltpu

N_NODES = 10000
BATCH = 8
IN_DIM = 2
UNITS = 64
FEAT = IN_DIM + UNITS
K_HOPS = 2
NUM_M = 2 * K_HOPS + 1
NBLK = 400
NGRID = N_NODES // NBLK


def _spmm_all(rows, cols, vals, x):
    def one(xb):
        return jax.ops.segment_sum(vals[:, None] * xb[cols], rows,
                                   num_segments=N_NODES)
    return jax.vmap(one)(x)


def _cheb_stack(rows, cols, vals, x0):
    x1 = _spmm_all(rows, cols, vals, x0)
    x2 = 2.0 * _spmm_all(rows, cols, vals, x1) - x0
    x0b = x1
    x1b = _spmm_all(cols, rows, vals, x0b)
    x2b = 2.0 * _spmm_all(cols, rows, vals, x1b) - x0b
    return jnp.stack([x0, x1, x2, x1b, x2b], axis=0)


def _gate1_body(x_ref, w_ref, b_ref, inp_ref, hx_ref, u_ref, x0p_ref, acc_ref):
    m = pl.program_id(2)
    prod = jnp.dot(x_ref[0, 0], w_ref[0], preferred_element_type=jnp.float32)

    @pl.when(m == 0)
    def _():
        acc_ref[...] = prod

    @pl.when(m > 0)
    def _():
        acc_ref[...] += prod

    @pl.when(m == NUM_M - 1)
    def _():
        val = jax.nn.sigmoid(acc_ref[...] + b_ref[0][None, :])
        r = val[:, :UNITS]
        u = val[:, UNITS:]
        u_ref[0] = u
        x0p_ref[0] = jnp.concatenate([inp_ref[0], r * hx_ref[0]], axis=1)


def _gate2_body(x_ref, w_ref, b_ref, hx_ref, u_ref, out_ref, acc_ref):
    m = pl.program_id(2)
    prod = jnp.dot(x_ref[0, 0], w_ref[0], preferred_element_type=jnp.float32)

    @pl.when(m == 0)
    def _():
        acc_ref[...] = prod

    @pl.when(m > 0)
    def _():
        acc_ref[...] += prod

    @pl.when(m == NUM_M - 1)
    def _():
        c = jnp.tanh(acc_ref[...] + b_ref[0][None, :])
        u = u_ref[0]
        out_ref[0] = u * hx_ref[0] + (1.0 - u) * c


@functools.partial(jax.jit, static_argnames=())
def _gate1(xstack, w_r, b2, inp_bn, hx_bn):
    grid = (NGRID, BATCH, NUM_M)
    return pl.pallas_call(
        _gate1_body,
        grid=grid,
        in_specs=[
            pl.BlockSpec((1, 1, NBLK, FEAT), lambda n, b, m: (m, b, n, 0)),
            pl.BlockSpec((1, FEAT, 2 * UNITS), lambda n, b, m: (m, 0, 0)),
            pl.BlockSpec((1, 2 * UNITS), lambda n, b, m: (0, 0)),
            pl.BlockSpec((1, NBLK, IN_DIM), lambda n, b, m: (b, n, 0)),
            pl.BlockSpec((1, NBLK, UNITS), lambda n, b, m: (b, n, 0)),
        ],
        out_specs=[
            pl.BlockSpec((1, NBLK, UNITS), lambda n, b, m: (b, n, 0)),
            pl.BlockSpec((1, NBLK, FEAT), lambda n, b, m: (b, n, 0)),
        ],
        out_shape=[
            jax.ShapeDtypeStruct((BATCH, N_NODES, UNITS), jnp.float32),
            jax.ShapeDtypeStruct((BATCH, N_NODES, FEAT), jnp.float32),
        ],
        scratch_shapes=[pltpu.VMEM((NBLK, 2 * UNITS), jnp.float32)],
    )(xstack, w_r, b2, inp_bn, hx_bn)


@functools.partial(jax.jit, static_argnames=())
def _gate2(xstack, w_r, b2, hx_bn, u_bn):
    grid = (NGRID, BATCH, NUM_M)
    return pl.pallas_call(
        _gate2_body,
        grid=grid,
        in_specs=[
            pl.BlockSpec((1, 1, NBLK, FEAT), lambda n, b, m: (m, b, n, 0)),
            pl.BlockSpec((1, FEAT, UNITS), lambda n, b, m: (m, 0, 0)),
            pl.BlockSpec((1, UNITS), lambda n, b, m: (0, 0)),
            pl.BlockSpec((1, NBLK, UNITS), lambda n, b, m: (b, n, 0)),
            pl.BlockSpec((1, NBLK, UNITS), lambda n, b, m: (b, n, 0)),
        ],
        out_specs=pl.BlockSpec((1, NBLK, UNITS), lambda n, b, m: (b, n, 0)),
        out_shape=jax.ShapeDtypeStruct((BATCH, N_NODES, UNITS), jnp.float32),
        scratch_shapes=[pltpu.VMEM((NBLK, UNITS), jnp.float32)],
    )(xstack, w_r, b2, hx_bn, u_bn)


def kernel(inputs, hx, rows, cols, vals, W_ru, b_ru, W_c, b_c):
    inp_bn = inputs.reshape(BATCH, N_NODES, IN_DIM)
    hx_bn = hx.reshape(BATCH, N_NODES, UNITS)
    x0 = jnp.concatenate([inp_bn, hx_bn], axis=2)

    w_ru_r = W_ru.reshape(FEAT, NUM_M, 2 * UNITS).transpose(1, 0, 2)
    w_c_r = W_c.reshape(FEAT, NUM_M, UNITS).transpose(1, 0, 2)
    b_ru2 = b_ru.reshape(1, 2 * UNITS)
    b_c2 = b_c.reshape(1, UNITS)

    xstack1 = _cheb_stack(rows, cols, vals, x0)
    u_bn, x0p = _gate1(xstack1, w_ru_r, b_ru2, inp_bn, hx_bn)

    xstack2 = _cheb_stack(rows, cols, vals, x0p)
    new_bn = _gate2(xstack2, w_c_r, b_c2, hx_bn, u_bn)
    return new_bn.reshape(BATCH, N_NODES * UNITS)


# speedup vs baseline: 21.4336x; 21.4336x over previous
import functools

import jax
import jax.numpy as jnp
import numpy as np
from jax import lax
from jax.experimental import pallas as pl
from jax.experimental.pallas import tpu as pltpu
from jax.experimental.pallas import tpu_sc as plsc

N_NODES = 10000
BATCH = 8
IN_DIM = 2
UNITS = 64
FEAT = IN_DIM + UNITS
NUM_M = 5
NBLK = 400
NGRID = N_NODES // NBLK

E_EDGES = 160000
NT = 16
EPT = E_EDGES // NT
BLK = 80
NBLK_E = EPT // BLK
DRB = 80
NDRAIN = N_NODES // DRB
DSUB = -(-NDRAIN // NT)

PFEAT = 80
ROW_OFFS = (0, 16, 32, 48, 64)
IDX_OFFS = (0, 16, 32, 48, 64)


def _diffuse(x0f, rows3, cols3, vals3):
    mesh = plsc.VectorSubcoreMesh(core_axis_name="c", subcore_axis_name="s")

    @functools.partial(
        pl.kernel,
        out_type=jax.ShapeDtypeStruct((4 * BATCH * N_NODES, PFEAT), jnp.float32),
        mesh=mesh,
        compiler_params=pltpu.CompilerParams(use_tc_tiling_on_sc=False),
        scratch_types=[
            pltpu.VMEM_SHARED((N_NODES, PFEAT), jnp.float32),
            pltpu.VMEM((NBLK_E, BLK), jnp.int32),
            pltpu.VMEM((NBLK_E, BLK), jnp.int32),
            pltpu.VMEM((NBLK_E, BLK), jnp.float32),
            pltpu.VMEM((NBLK_E, BLK), jnp.int32),
            pltpu.VMEM((BLK, PFEAT), jnp.float32),
            pltpu.VMEM((BLK, PFEAT), jnp.float32),
            pltpu.VMEM((BLK, PFEAT), jnp.float32),
            pltpu.VMEM((BLK, PFEAT), jnp.float32),
            pltpu.VMEM((DRB, PFEAT), jnp.float32),
            pltpu.VMEM((DRB, PFEAT), jnp.float32),
            pltpu.SemaphoreType.DMA,
            pltpu.SemaphoreType.DMA,
            pltpu.SemaphoreType.DMA,
            pltpu.SemaphoreType.DMA,
        ],
    )
    def diffuse_kernel(x0_hbm, rows_hbm, cols_hbm, vals_hbm, out_hbm,
                       acc, rows_t, cols_t, vals_t, gidx,
                       gb0, gb1, sb0, sb1, dbuf, mbuf,
                       sg0, sg1, ss0, ss1):
        c = lax.axis_index("c")
        s = lax.axis_index("s")

        pltpu.sync_copy(rows_hbm.at[s], rows_t)
        pltpu.sync_copy(cols_hbm.at[s], cols_t)
        pltpu.sync_copy(vals_hbm.at[s], vals_t)

        zv = lax.broadcast(jnp.float32(0.0), (16,))
        iota16 = lax.iota(jnp.int32, 16)
        izero = iota16 - iota16

        def scale_block(gb, sb, blk):
            @pl.loop(0, BLK, step=16)
            def _(i0):
                vv = vals_t[blk, pl.ds(i0, 16)]
                for j in range(16):
                    vj = jnp.take_along_axis(vv, izero + j, axis=0,
                                             mode="promise_in_bounds")
                    for off in ROW_OFFS:
                        sb[i0 + j, pl.ds(off, 16)] = (
                            gb[i0 + j, pl.ds(off, 16)] * vj)

        def do_step(src_hbm, src_base, gsel, ssel, out_base, minus):
            @pl.loop(0, DRB)
            def _(i):
                for off in ROW_OFFS:
                    dbuf[i, pl.ds(off, 16)] = zv

            vb = lax.broadcast(src_base, (16,))

            @pl.loop(0, NBLK_E)
            def _(k):
                for off in IDX_OFFS:
                    gidx[k, pl.ds(off, 16)] = gsel[k, pl.ds(off, 16)] + vb

            for j in range(DSUB):
                blk_id = s + j * NT

                @pl.when(blk_id < NDRAIN)
                def _():
                    pltpu.sync_copy(dbuf, acc.at[pl.ds(blk_id * DRB, DRB)])
            plsc.subcore_barrier()

            @pl.loop(0, NBLK_E - 1, step=2)
            def _(g):
                da = pltpu.async_copy(src_hbm.at[gidx.at[g]], gb0, sg0)
                db = pltpu.async_copy(src_hbm.at[gidx.at[g + 1]], gb1, sg1)
                da.wait()
                scale_block(gb0, sb0, g)
                sa = pltpu.async_copy(sb0, acc.at[ssel.at[g]], ss0, add=True)
                db.wait()
                scale_block(gb1, sb1, g + 1)
                sa.wait()
                sb = pltpu.async_copy(sb1, acc.at[ssel.at[g + 1]], ss1,
                                      add=True)
                sb.wait()

            dl = pltpu.async_copy(src_hbm.at[gidx.at[NBLK_E - 1]], gb0, sg0)
            dl.wait()
            scale_block(gb0, sb0, NBLK_E - 1)
            sl = pltpu.async_copy(sb0, acc.at[ssel.at[NBLK_E - 1]], ss0,
                                  add=True)
            sl.wait()
            plsc.subcore_barrier()

            for j in range(DSUB):
                blk_id = s + j * NT

                @pl.when(blk_id < NDRAIN)
                def _():
                    row0 = blk_id * DRB
                    pltpu.sync_copy(acc.at[pl.ds(row0, DRB)], dbuf)
                    if minus is not None:
                        m_hbm, m_base = minus
                        pltpu.sync_copy(
                            m_hbm.at[pl.ds(m_base + row0, DRB)], mbuf)

                        @pl.loop(0, DRB)
                        def _(i):
                            for off in ROW_OFFS:
                                v = dbuf[i, pl.ds(off, 16)]
                                dbuf[i, pl.ds(off, 16)] = (
                                    v + v - mbuf[i, pl.ds(off, 16)])
                    pltpu.sync_copy(
                        dbuf, out_hbm.at[pl.ds(out_base + row0, DRB)])

        @pl.loop(0, 4)
        def _(bl):
            b = c * 4 + bl
            x0b = b * N_NODES
            x1b = (0 * BATCH + b) * N_NODES
            x2b = (1 * BATCH + b) * N_NODES
            x1bb = (2 * BATCH + b) * N_NODES
            x2bb = (3 * BATCH + b) * N_NODES
            do_step(x0_hbm, x0b, cols_t, rows_t, x1b, None)
            do_step(out_hbm, x1b, cols_t, rows_t, x2b, (x0_hbm, x0b))
            do_step(out_hbm, x1b, rows_t, cols_t, x1bb, None)
            do_step(out_hbm, x1bb, rows_t, cols_t, x2bb, (out_hbm, x1b))

    return diffuse_kernel(x0f, rows3, cols3, vals3)


def _gate1_body(x0_ref, xs_ref, w_ref, b_ref, inp_ref, hx_ref,
                u_ref, x0p_ref, acc_ref):
    m = pl.program_id(2)

    @pl.when(m == 0)
    def _():
        acc_ref[...] = jnp.dot(x0_ref[0], w_ref[0],
                               preferred_element_type=jnp.float32)

    @pl.when(m > 0)
    def _():
        acc_ref[...] += jnp.dot(xs_ref[0, 0], w_ref[0],
                                preferred_element_type=jnp.float32)

    @pl.when(m == NUM_M - 1)
    def _():
        val = jax.nn.sigmoid(acc_ref[...] + b_ref[0][None, :])
        r = val[:, :UNITS]
        u = val[:, UNITS:]
        u_ref[0] = u
        x0p_ref[0] = jnp.concatenate(
            [inp_ref[0], r * hx_ref[0],
             jnp.zeros((NBLK, PFEAT - FEAT), jnp.float32)], axis=1)


def _gate2_body(x0_ref, xs_ref, w_ref, b_ref, hx_ref, u_ref, out_ref, acc_ref):
    m = pl.program_id(2)

    @pl.when(m == 0)
    def _():
        acc_ref[...] = jnp.dot(x0_ref[0], w_ref[0],
                               preferred_element_type=jnp.float32)

    @pl.when(m > 0)
    def _():
        acc_ref[...] += jnp.dot(xs_ref[0, 0], w_ref[0],
                                preferred_element_type=jnp.float32)

    @pl.when(m == NUM_M - 1)
    def _():
        cc = jnp.tanh(acc_ref[...] + b_ref[0][None, :])
        u = u_ref[0]
        out_ref[0] = u * hx_ref[0] + (1.0 - u) * cc


def _xs_index(n, b, m):
    return (jnp.where(m < 1, 0, m - 1), b, n, 0)


def _gate1(x0_bn, xs, w_r, b2, inp_bn, hx_bn):
    grid = (NGRID, BATCH, NUM_M)
    return pl.pallas_call(
        _gate1_body,
        grid=grid,
        in_specs=[
            pl.BlockSpec((1, NBLK, PFEAT), lambda n, b, m: (b, n, 0)),
            pl.BlockSpec((1, 1, NBLK, PFEAT), _xs_index),
            pl.BlockSpec((1, PFEAT, 2 * UNITS), lambda n, b, m: (m, 0, 0)),
            pl.BlockSpec((1, 2 * UNITS), lambda n, b, m: (0, 0)),
            pl.BlockSpec((1, NBLK, IN_DIM), lambda n, b, m: (b, n, 0)),
            pl.BlockSpec((1, NBLK, UNITS), lambda n, b, m: (b, n, 0)),
        ],
        out_specs=[
            pl.BlockSpec((1, NBLK, UNITS), lambda n, b, m: (b, n, 0)),
            pl.BlockSpec((1, NBLK, PFEAT), lambda n, b, m: (b, n, 0)),
        ],
        out_shape=[
            jax.ShapeDtypeStruct((BATCH, N_NODES, UNITS), jnp.float32),
            jax.ShapeDtypeStruct((BATCH, N_NODES, PFEAT), jnp.float32),
        ],
        scratch_shapes=[pltpu.VMEM((NBLK, 2 * UNITS), jnp.float32)],
    )(x0_bn, xs, w_r, b2, inp_bn, hx_bn)


def _gate2(x0_bn, xs, w_r, b2, hx_bn, u_bn):
    grid = (NGRID, BATCH, NUM_M)
    return pl.pallas_call(
        _gate2_body,
        grid=grid,
        in_specs=[
            pl.BlockSpec((1, NBLK, PFEAT), lambda n, b, m: (b, n, 0)),
            pl.BlockSpec((1, 1, NBLK, PFEAT), _xs_index),
            pl.BlockSpec((1, PFEAT, UNITS), lambda n, b, m: (m, 0, 0)),
            pl.BlockSpec((1, UNITS), lambda n, b, m: (0, 0)),
            pl.BlockSpec((1, NBLK, UNITS), lambda n, b, m: (b, n, 0)),
            pl.BlockSpec((1, NBLK, UNITS), lambda n, b, m: (b, n, 0)),
        ],
        out_specs=pl.BlockSpec((1, NBLK, UNITS), lambda n, b, m: (b, n, 0)),
        out_shape=jax.ShapeDtypeStruct((BATCH, N_NODES, UNITS), jnp.float32),
        scratch_shapes=[pltpu.VMEM((NBLK, UNITS), jnp.float32)],
    )(x0_bn, xs, w_r, b2, hx_bn, u_bn)


def kernel(inputs, hx, rows, cols, vals, W_ru, b_ru, W_c, b_c):
    inp_bn = inputs.reshape(BATCH, N_NODES, IN_DIM)
    hx_bn = hx.reshape(BATCH, N_NODES, UNITS)
    x0_bn = jnp.concatenate(
        [inp_bn, hx_bn,
         jnp.zeros((BATCH, N_NODES, PFEAT - FEAT), jnp.float32)],
        axis=2)

    rows3 = rows.reshape(NT, NBLK_E, BLK)
    cols3 = cols.reshape(NT, NBLK_E, BLK)
    vals3 = vals.reshape(NT, NBLK_E, BLK)

    w_ru_r = jnp.pad(W_ru.reshape(FEAT, NUM_M, 2 * UNITS),
                     ((0, PFEAT - FEAT), (0, 0), (0, 0))).transpose(1, 0, 2)
    w_c_r = jnp.pad(W_c.reshape(FEAT, NUM_M, UNITS),
                    ((0, PFEAT - FEAT), (0, 0), (0, 0))).transpose(1, 0, 2)
    b_ru2 = b_ru.reshape(1, 2 * UNITS)
    b_c2 = b_c.reshape(1, UNITS)

    xs1 = _diffuse(x0_bn.reshape(BATCH * N_NODES, PFEAT), rows3, cols3, vals3)
    xs1 = xs1.reshape(4, BATCH, N_NODES, PFEAT)
    u_bn, x0p = _gate1(x0_bn, xs1, w_ru_r, b_ru2, inp_bn, hx_bn)

    xs2 = _diffuse(x0p.reshape(BATCH * N_NODES, PFEAT), rows3, cols3, vals3)
    xs2 = xs2.reshape(4, BATCH, N_NODES, PFEAT)
    new_bn = _gate2(x0p, xs2, w_c_r, b_c2, hx_bn, u_bn)
    return new_bn.reshape(BATCH, N_NODES * UNITS)
